# baseline (speedup 1.0000x reference)
import functools

import jax
import jax.numpy as jnp
from jax.experimental import pallas as pl


def _moe_chunk(xb, wg, bg, w1, b1, w2, b2, expand, E, F):
    logits = jnp.dot(xb, wg, preferred_element_type=jnp.float32) + bg
    m = jnp.max(logits, axis=-1, keepdims=True)
    p = jnp.exp(logits - m)
    p = p / jnp.sum(p, axis=-1, keepdims=True)

    i1 = jnp.argmax(p, axis=-1)[:, None]
    top1 = jnp.max(p, axis=-1, keepdims=True)
    cols = jax.lax.broadcasted_iota(jnp.int32, p.shape, 1)
    p2 = jnp.where(cols == i1, -jnp.inf, p)
    i2 = jnp.argmax(p2, axis=-1)[:, None]
    top2 = jnp.max(p2, axis=-1, keepdims=True)
    denom = top1 + top2
    c = (jnp.where(cols == i1, top1, 0.0) + jnp.where(cols == i2, top2, 0.0)) / denom

    cw = jnp.dot(c.astype(jnp.bfloat16), expand, preferred_element_type=jnp.float32)
    h = jnp.dot(xb.astype(jnp.bfloat16), w1, preferred_element_type=jnp.float32)
    h = (jnp.maximum(h + b1, 0.0) * cw).astype(jnp.bfloat16)
    y = jnp.dot(h, w2, preferred_element_type=jnp.float32)
    return y + jnp.dot(c, b2, preferred_element_type=jnp.float32)


def _moe_block(x_ref, wg_ref, bg_ref, w1_ref, b1_ref, w2_ref, b2_ref,
               exp_ref, o_ref, *, num_experts, expert_size, chunk):
    E, F = num_experts, expert_size
    M = x_ref.shape[0]
    for s in range(0, M, chunk):
        o_ref[pl.ds(s, chunk), :] = _moe_chunk(
            x_ref[pl.ds(s, chunk), :], wg_ref[...], bg_ref[...], w1_ref[...],
            b1_ref[...], w2_ref[...], b2_ref[...], exp_ref[...], E, F)


def kernel(x, Wg, bg, W1, b1, W2, b2):
    B, S, H = x.shape
    E, _, F = W1.shape
    N = B * S
    xf = x.reshape(N, H)
    M = 512
    CHUNK = 512
    grid = (N // M,)

    W1a = W1.transpose(1, 0, 2).reshape(H, E * F).astype(jnp.bfloat16)
    W2a = W2.reshape(E * F, H).astype(jnp.bfloat16)
    b1a = b1.reshape(1, E * F)
    expand = jnp.repeat(jnp.eye(E, dtype=jnp.bfloat16), F, axis=1)

    out = pl.pallas_call(
        functools.partial(_moe_block, num_experts=E, expert_size=F, chunk=CHUNK),
        grid=grid,
        in_specs=[
            pl.BlockSpec((M, H), lambda i: (i, 0)),
            pl.BlockSpec((H, E), lambda i: (0, 0)),
            pl.BlockSpec((1, E), lambda i: (0, 0)),
            pl.BlockSpec((H, E * F), lambda i: (0, 0)),
            pl.BlockSpec((1, E * F), lambda i: (0, 0)),
            pl.BlockSpec((E * F, H), lambda i: (0, 0)),
            pl.BlockSpec((E, H), lambda i: (0, 0)),
            pl.BlockSpec((E, E * F), lambda i: (0, 0)),
        ],
        out_specs=pl.BlockSpec((M, H), lambda i: (i, 0)),
        out_shape=jax.ShapeDtypeStruct((N, H), jnp.float32),
    )(xf, Wg, bg.reshape(1, E), W1a, b1a, W2a, b2, expand)
    return out.reshape(B, S, H)


# speedup vs baseline: 1.0100x; 1.0100x over previous
import functools

import jax
import jax.numpy as jnp
from jax.experimental import pallas as pl
from jax.experimental.pallas import tpu as pltpu


def _moe_block(x_ref, wg_ref, bg_ref, w1_ref, b1_ref, w2_ref, b2_ref, o_ref,
               w1s, w2s, exps, *, num_experts, expert_size):
    E, F = num_experts, expert_size

    @pl.when(pl.program_id(0) == 0)
    def _init():
        for e in range(E):
            w1s[:, e * F:(e + 1) * F] = w1_ref[e].astype(jnp.bfloat16)
            w2s[e * F:(e + 1) * F, :] = w2_ref[e].astype(jnp.bfloat16)
        rows = jax.lax.broadcasted_iota(jnp.int32, (E, E * F), 0)
        cols = jax.lax.broadcasted_iota(jnp.int32, (E, E * F), 1)
        block = (cols >= rows * F) & (cols < (rows + 1) * F)
        exps[...] = block.astype(jnp.bfloat16)

    xb = x_ref[...]
    logits = jnp.dot(xb, wg_ref[...], preferred_element_type=jnp.float32)
    logits = logits + bg_ref[...]
    m = jnp.max(logits, axis=-1, keepdims=True)
    p = jnp.exp(logits - m)
    p = p / jnp.sum(p, axis=-1, keepdims=True)

    i1 = jnp.argmax(p, axis=-1)[:, None]
    top1 = jnp.max(p, axis=-1, keepdims=True)
    cols = jax.lax.broadcasted_iota(jnp.int32, p.shape, 1)
    p2 = jnp.where(cols == i1, -jnp.inf, p)
    i2 = jnp.argmax(p2, axis=-1)[:, None]
    top2 = jnp.max(p2, axis=-1, keepdims=True)
    denom = top1 + top2
    c = (jnp.where(cols == i1, top1, 0.0) + jnp.where(cols == i2, top2, 0.0)) / denom

    cw = jnp.dot(c, exps[...], preferred_element_type=jnp.float32)
    h = jnp.dot(xb.astype(jnp.bfloat16), w1s[...],
                preferred_element_type=jnp.float32)
    h = (jnp.maximum(h + b1_ref[...], 0.0) * cw).astype(jnp.bfloat16)
    y = jnp.dot(h, w2s[...], preferred_element_type=jnp.float32)
    y = y + jnp.dot(c, b2_ref[...], preferred_element_type=jnp.float32)
    o_ref[...] = y


def kernel(x, Wg, bg, W1, b1, W2, b2):
    B, S, H = x.shape
    E, _, F = W1.shape
    N = B * S
    xf = x.reshape(N, H)
    M = 512
    grid = (N // M,)

    out = pl.pallas_call(
        functools.partial(_moe_block, num_experts=E, expert_size=F),
        grid=grid,
        in_specs=[
            pl.BlockSpec((M, H), lambda i: (i, 0)),
            pl.BlockSpec((H, E), lambda i: (0, 0)),
            pl.BlockSpec((1, E), lambda i: (0, 0)),
            pl.BlockSpec((E, H, F), lambda i: (0, 0, 0)),
            pl.BlockSpec((1, E * F), lambda i: (0, 0)),
            pl.BlockSpec((E, F, H), lambda i: (0, 0, 0)),
            pl.BlockSpec((E, H), lambda i: (0, 0)),
        ],
        out_specs=pl.BlockSpec((M, H), lambda i: (i, 0)),
        out_shape=jax.ShapeDtypeStruct((N, H), jnp.float32),
        scratch_shapes=[
            pltpu.VMEM((H, E * F), jnp.bfloat16),
            pltpu.VMEM((E * F, H), jnp.bfloat16),
            pltpu.VMEM((E, E * F), jnp.bfloat16),
        ],
    )(xf, Wg, bg.reshape(1, E), W1, b1.reshape(1, E * F), W2, b2)
    return out.reshape(B, S, H)


# speedup vs baseline: 1.0219x; 1.0118x over previous
import functools

import jax
import jax.numpy as jnp
from jax.experimental import pallas as pl


def _moe_block(x_ref, wg_ref, bg_ref, w1_ref, b1_ref, w2_ref, b2_ref, o_ref,
               *, num_experts, expert_size):
    E, F = num_experts, expert_size
    xb = x_ref[...]
    logits = jnp.dot(xb, wg_ref[...], preferred_element_type=jnp.float32)
    logits = logits + bg_ref[...]
    m = jnp.max(logits, axis=-1, keepdims=True)
    p = jnp.exp(logits - m)
    p = p / jnp.sum(p, axis=-1, keepdims=True)

    i1 = jnp.argmax(p, axis=-1)[:, None]
    top1 = jnp.max(p, axis=-1, keepdims=True)
    cols = jax.lax.broadcasted_iota(jnp.int32, p.shape, 1)
    p2 = jnp.where(cols == i1, -jnp.inf, p)
    i2 = jnp.argmax(p2, axis=-1)[:, None]
    top2 = jnp.max(p2, axis=-1, keepdims=True)
    denom = top1 + top2
    c = (jnp.where(cols == i1, top1, 0.0) + jnp.where(cols == i2, top2, 0.0)) / denom

    xb16 = xb.astype(jnp.bfloat16)
    acc = jnp.dot(c, b2_ref[...], preferred_element_type=jnp.float32)
    for e in range(E):
        h = jnp.dot(xb16, w1_ref[e], preferred_element_type=jnp.float32)
        h = jnp.maximum(h + b1_ref[e], 0.0) * c[:, e:e + 1]
        acc = acc + jnp.dot(h.astype(jnp.bfloat16), w2_ref[e],
                            preferred_element_type=jnp.float32)
    o_ref[...] = acc


def kernel(x, Wg, bg, W1, b1, W2, b2):
    B, S, H = x.shape
    E, _, F = W1.shape
    N = B * S
    xf = x.reshape(N, H)
    M = 512
    grid = (N // M,)

    W1a = W1.astype(jnp.bfloat16)
    W2a = W2.astype(jnp.bfloat16)

    out = pl.pallas_call(
        functools.partial(_moe_block, num_experts=E, expert_size=F),
        grid=grid,
        in_specs=[
            pl.BlockSpec((M, H), lambda i: (i, 0)),
            pl.BlockSpec((H, E), lambda i: (0, 0)),
            pl.BlockSpec((1, E), lambda i: (0, 0)),
            pl.BlockSpec((E, H, F), lambda i: (0, 0, 0)),
            pl.BlockSpec((E, 1, F), lambda i: (0, 0, 0)),
            pl.BlockSpec((E, F, H), lambda i: (0, 0, 0)),
            pl.BlockSpec((E, H), lambda i: (0, 0)),
        ],
        out_specs=pl.BlockSpec((M, H), lambda i: (i, 0)),
        out_shape=jax.ShapeDtypeStruct((N, H), jnp.float32),
    )(xf, Wg, bg.reshape(1, E), W1a, b1.reshape(E, 1, F), W2a, b2)
    return out.reshape(B, S, H)
